# baseline (speedup 1.0000x reference)
import jax
import jax.numpy as jnp
from jax import lax
from jax.experimental import pallas as pl
from jax.experimental.pallas import tpu as pltpu
from jax.experimental.pallas import tpu_sc as plsc

T = 1000
LATENT_DIM = 128
BATCH = 16384

_info = plsc.get_sparse_core_info()
_NC, _NS = _info.num_cores, _info.num_subcores
_NW = _NC * _NS
_CHUNK = 128
_ROWS_PER_W = BATCH // _NW
_CHUNKS_PER_W = _ROWS_PER_W // _CHUNK


def _gather_body(t_hbm, table_hbm, out_hbm, idx_v, rows_v, gsems, ssem):
    wid = lax.axis_index("s") * _NC + lax.axis_index("c")
    pltpu.sync_copy(t_hbm.at[pl.ds(wid * _CHUNKS_PER_W, _CHUNKS_PER_W)], idx_v)
    gathers = []
    for j in range(_CHUNKS_PER_W):
        gathers.append(
            pltpu.async_copy(
                table_hbm.at[idx_v.at[j]],
                rows_v.at[pl.ds(j * _CHUNK, _CHUNK)],
                gsems.at[j],
            )
        )
    stores = []
    for j in range(_CHUNKS_PER_W):
        gathers[j].wait()
        stores.append(
            pltpu.async_copy(
                rows_v.at[pl.ds(j * _CHUNK, _CHUNK)],
                out_hbm.at[pl.ds(wid * _ROWS_PER_W + j * _CHUNK, _CHUNK)],
                ssem,
            )
        )
    for d in stores:
        d.wait()


def kernel(t, table):
    t_2d = t.astype(jnp.int32).reshape(BATCH // _CHUNK, _CHUNK)
    mesh = plsc.VectorSubcoreMesh(core_axis_name="c", subcore_axis_name="s")
    return pl.kernel(
        _gather_body,
        out_type=jax.ShapeDtypeStruct((BATCH, LATENT_DIM), jnp.float32),
        mesh=mesh,
        scratch_types=[
            pltpu.VMEM((_CHUNKS_PER_W, _CHUNK), jnp.int32),
            pltpu.VMEM((_ROWS_PER_W, LATENT_DIM), jnp.float32),
            pltpu.SemaphoreType.DMA((_CHUNKS_PER_W,)),
            pltpu.SemaphoreType.DMA,
        ],
    )(t_2d, table)


# speedup vs baseline: 1.0304x; 1.0304x over previous
import jax
import jax.numpy as jnp
from jax import lax
from jax.experimental import pallas as pl
from jax.experimental.pallas import tpu as pltpu
from jax.experimental.pallas import tpu_sc as plsc

T = 1000
LATENT_DIM = 128
BATCH = 16384

_info = plsc.get_sparse_core_info()
_NC, _NS = _info.num_cores, _info.num_subcores
_NW = _NC * _NS
_CHUNK = 128
_ROWS_PER_W = BATCH // _NW
_CHUNKS_PER_W = _ROWS_PER_W // _CHUNK


def _gather_body(t_hbm, table_hbm, out_hbm, idx_v, rows_v, sem):
    wid = lax.axis_index("s") * _NC + lax.axis_index("c")
    base = wid * _ROWS_PER_W
    pltpu.sync_copy(t_hbm.at[pl.ds(base, _ROWS_PER_W)], idx_v)
    descs = []
    for j in range(_CHUNKS_PER_W):
        descs.append(
            pltpu.async_copy(
                table_hbm.at[idx_v.at[pl.ds(j * _CHUNK, _CHUNK)]],
                rows_v.at[pl.ds(j * _CHUNK, _CHUNK)],
                sem,
            )
        )
    for d in descs:
        d.wait()
    pltpu.sync_copy(rows_v, out_hbm.at[pl.ds(base, _ROWS_PER_W)])


def kernel(t, table):
    mesh = plsc.VectorSubcoreMesh(core_axis_name="c", subcore_axis_name="s")
    return pl.kernel(
        _gather_body,
        out_type=jax.ShapeDtypeStruct((BATCH, LATENT_DIM), jnp.float32),
        mesh=mesh,
        scratch_types=[
            pltpu.VMEM((_ROWS_PER_W,), jnp.int32),
            pltpu.VMEM((_ROWS_PER_W, LATENT_DIM), jnp.float32),
            pltpu.SemaphoreType.DMA,
        ],
    )(t, table)


# speedup vs baseline: 1.1794x; 1.1446x over previous
import jax
import jax.numpy as jnp
from jax import lax
from jax.experimental import pallas as pl
from jax.experimental.pallas import tpu as pltpu

T = 1000
TPAD = 1024
LATENT_DIM = 128
BATCH = 16384
BLK = 1024


def _tc_body(t_ref, tab_hi_ref, tab_lo_ref, out_ref):
    t_blk = t_ref[0, 0]
    ks = jax.lax.broadcasted_iota(jnp.int32, (BLK, TPAD), 1)
    oh = (t_blk[:, None] == ks).astype(jnp.bfloat16)
    acc = jnp.dot(oh, tab_hi_ref[...], preferred_element_type=jnp.float32)
    acc = acc + jnp.dot(oh, tab_lo_ref[...], preferred_element_type=jnp.float32)
    out_ref[...] = acc


def kernel(t, table):
    tab_hi = table.astype(jnp.bfloat16)
    tab_lo = (table - tab_hi.astype(jnp.float32)).astype(jnp.bfloat16)
    pad = [(0, TPAD - T), (0, 0)]
    tab_hi = jnp.pad(tab_hi, pad)
    tab_lo = jnp.pad(tab_lo, pad)
    t_3d = t.astype(jnp.int32).reshape(BATCH // BLK, 1, BLK)
    grid = (BATCH // BLK,)
    return pl.pallas_call(
        _tc_body,
        grid=grid,
        in_specs=[
            pl.BlockSpec((1, 1, BLK), lambda i: (i, 0, 0)),
            pl.BlockSpec((TPAD, LATENT_DIM), lambda i: (0, 0)),
            pl.BlockSpec((TPAD, LATENT_DIM), lambda i: (0, 0)),
        ],
        out_specs=pl.BlockSpec((BLK, LATENT_DIM), lambda i: (i, 0)),
        out_shape=jax.ShapeDtypeStruct((BATCH, LATENT_DIM), jnp.float32),
    )(t_3d, tab_hi, tab_lo)
